# Initial kernel scaffold; baseline (speedup 1.0000x reference)
import jax
import jax.numpy as jnp
from jax.experimental import pallas as pl


def kernel(x, Wg, W1, b1, W2, b2):
    raise NotImplementedError("write your pallas kernel here")


# speedup vs baseline: 1.8306x; 1.8306x over previous
import functools

import jax
import jax.numpy as jnp
from jax import lax
from jax.experimental import pallas as pl
from jax.experimental.pallas import tpu as pltpu
from jax.experimental.pallas import tpu_sc as plsc

NT = 2048
DM = 768
DFF = 3072
NE = 8
LN = 128
BLK = 256
NB = 24
SP = NB * BLK
NW = 32
TPW = NT // NW


def _router_body(x_ref, wg_ref, dst0_ref, dst1_ref, g0_ref, bexp_ref,
                 nact_ref, loss_ref):
    xv = x_ref[...]
    wg = wg_ref[...]
    logits = jnp.dot(xv, wg, preferred_element_type=jnp.float32)

    lane = lax.broadcasted_iota(jnp.int32, (NT, LN), 1)
    valid = lane < NE
    neg = jnp.float32(-1e30)
    lg = jnp.where(valid, logits, neg)

    m1 = jnp.max(lg, axis=1, keepdims=True)
    i1 = jnp.min(jnp.where((lg == m1) & valid, lane, LN), axis=1,
                 keepdims=True)
    lg2 = jnp.where(lane == i1, neg, lg)
    m2 = jnp.max(lg2, axis=1, keepdims=True)
    i2 = jnp.min(jnp.where((lg2 == m2) & valid, lane, LN), axis=1,
                 keepdims=True)

    g0 = 1.0 / (1.0 + jnp.exp(m2 - m1))

    oh1 = ((lane == i1) & valid).astype(jnp.float32)
    oh2 = ((lane == i2) & valid).astype(jnp.float32)
    both = oh1 + oh2

    s = both
    k = 1
    while k < NT:
        pad = jnp.zeros((k, LN), jnp.float32)
        s = s + jnp.concatenate([pad, s[:-k, :]], axis=0)
        k *= 2
    s_excl = s - both
    counts = s[NT - 1:NT, :]

    nbk = jnp.floor((counts + jnp.float32(BLK - 1)) *
                    jnp.float32(1.0 / BLK))
    r128 = lax.broadcasted_iota(jnp.int32, (LN, LN), 0)
    c128 = lax.broadcasted_iota(jnp.int32, (LN, LN), 1)
    m_lt = (r128 < c128).astype(jnp.float32)
    pstartb = jnp.dot(nbk, m_lt, preferred_element_type=jnp.float32)
    pstart = pstartb * jnp.float32(BLK)
    nact = jnp.sum(nbk)

    ps_b = jnp.broadcast_to(pstart, (NT, LN))
    dst0 = jnp.sum(oh1 * (ps_b + s_excl), axis=1, keepdims=True)
    dst1 = jnp.sum(oh2 * (ps_b + s_excl), axis=1, keepdims=True)
    dst0_ref[...] = jnp.broadcast_to(dst0, (NT, LN)).astype(jnp.int32)
    dst1_ref[...] = jnp.broadcast_to(dst1, (NT, LN)).astype(jnp.int32)
    g0_ref[...] = jnp.broadcast_to(g0, (NT, LN))

    psb_rows = jnp.broadcast_to(pstartb, (LN, LN))
    kmat = ((psb_rows <= r128.astype(jnp.float32)) &
            (c128 < NE)).astype(jnp.float32)
    bexp = jnp.sum(kmat, axis=1, keepdims=True) - 1.0
    bexp_ref[...] = jnp.broadcast_to(bexp, (LN, LN)).astype(jnp.int32)
    nact_ref[...] = jnp.full((8, LN), nact, jnp.float32).astype(jnp.int32)

    p_un = jnp.where(valid, jnp.exp(lg - m1), 0.0)
    probs = p_un / jnp.sum(p_un, axis=1, keepdims=True)
    mean_prob = jnp.sum(probs, axis=0, keepdims=True) * jnp.float32(1.0 / NT)
    frac = counts * jnp.float32(1.0 / NT)
    loss = jnp.float32(NE) * jnp.sum(frac * mean_prob)
    loss_ref[...] = jnp.full((8, LN), loss, jnp.float32)


def _router(x, wgp, interpret=False):
    return pl.pallas_call(
        _router_body,
        out_shape=(
            jax.ShapeDtypeStruct((NT, LN), jnp.int32),
            jax.ShapeDtypeStruct((NT, LN), jnp.int32),
            jax.ShapeDtypeStruct((NT, LN), jnp.float32),
            jax.ShapeDtypeStruct((LN, LN), jnp.int32),
            jax.ShapeDtypeStruct((8, LN), jnp.int32),
            jax.ShapeDtypeStruct((8, LN), jnp.float32),
        ),
        interpret=interpret,
    )(x, wgp)


def _dispatch_body(x_hbm, dst0_hbm, dst1_hbm, xg_hbm,
                   idx0_v, idx1_v, rows_v, sem0, sem1):
    wid = lax.axis_index("s") * 2 + lax.axis_index("c")
    base = wid * TPW
    pltpu.sync_copy(dst0_hbm.at[pl.ds(base, TPW)], idx0_v)
    pltpu.sync_copy(dst1_hbm.at[pl.ds(base, TPW)], idx1_v)
    pltpu.sync_copy(x_hbm.at[pl.ds(base, TPW), :], rows_v)
    c0 = pltpu.async_copy(rows_v, xg_hbm.at[idx0_v], sem0)
    c1 = pltpu.async_copy(rows_v, xg_hbm.at[idx1_v], sem1)
    c0.wait()
    c1.wait()


def _dispatch(x, dst0, dst1):
    mesh = plsc.VectorSubcoreMesh(core_axis_name="c", subcore_axis_name="s")
    f = pl.kernel(
        _dispatch_body,
        out_type=jax.ShapeDtypeStruct((SP, DM), jnp.float32),
        mesh=mesh,
        scratch_types=[
            pltpu.VMEM((TPW,), jnp.int32),
            pltpu.VMEM((TPW,), jnp.int32),
            pltpu.VMEM((TPW, DM), jnp.float32),
            pltpu.SemaphoreType.DMA,
            pltpu.SemaphoreType.DMA,
        ],
    )
    return f(x, dst0, dst1)


def _experts_body(bexp_ref, nact_ref, xg_ref, w1_ref, b1_ref, w2_ref, b2_ref,
                  y_ref):
    b = pl.program_id(0)

    @pl.when(b < nact_ref[0])
    def _():
        xb = xg_ref[...].astype(jnp.bfloat16)
        w1 = w1_ref[0].astype(jnp.bfloat16)
        h = jnp.dot(xb, w1, preferred_element_type=jnp.float32)
        h = h + b1_ref[0]
        h = jax.nn.gelu(h)
        w2 = w2_ref[0].astype(jnp.bfloat16)
        y = jnp.dot(h.astype(jnp.bfloat16), w2,
                    preferred_element_type=jnp.float32)
        y_ref[...] = y + b2_ref[0]


def _experts(bexp, nact, xg, w1, b1, w2, b2, interpret=False):
    grid_spec = pltpu.PrefetchScalarGridSpec(
        num_scalar_prefetch=2,
        grid=(NB,),
        in_specs=[
            pl.BlockSpec((BLK, DM), lambda b, be, na: (b, 0)),
            pl.BlockSpec((1, DM, DFF), lambda b, be, na: (be[b], 0, 0)),
            pl.BlockSpec((1, 1, DFF), lambda b, be, na: (be[b], 0, 0)),
            pl.BlockSpec((1, DFF, DM), lambda b, be, na: (be[b], 0, 0)),
            pl.BlockSpec((1, 1, DM), lambda b, be, na: (be[b], 0, 0)),
        ],
        out_specs=pl.BlockSpec((BLK, DM), lambda b, be, na: (b, 0)),
    )
    return pl.pallas_call(
        _experts_body,
        grid_spec=grid_spec,
        out_shape=jax.ShapeDtypeStruct((SP, DM), jnp.float32),
        interpret=interpret,
    )(bexp, nact, xg, w1, b1.reshape(NE, 1, DFF), w2, b2.reshape(NE, 1, DM))


def _combine_body(y_hbm, dst0_hbm, dst1_hbm, g0_hbm, out_hbm,
                  idx0_v, idx1_v, g0_v, rows0_v, rows1_v, sem0, sem1):
    wid = lax.axis_index("s") * 2 + lax.axis_index("c")
    base = wid * TPW
    pltpu.sync_copy(dst0_hbm.at[pl.ds(base, TPW)], idx0_v)
    pltpu.sync_copy(dst1_hbm.at[pl.ds(base, TPW)], idx1_v)
    pltpu.sync_copy(g0_hbm.at[pl.ds(base, TPW), :], g0_v)
    c0 = pltpu.async_copy(y_hbm.at[idx0_v], rows0_v, sem0)
    c1 = pltpu.async_copy(y_hbm.at[idx1_v], rows1_v, sem1)
    c0.wait()
    c1.wait()

    def row_body(r, carry):
        ga = g0_v[r, :]
        gb = 1.0 - ga
        for j in range(DM // 16):
            a = rows0_v[r, pl.ds(j * 16, 16)]
            bv = rows1_v[r, pl.ds(j * 16, 16)]
            rows0_v[r, pl.ds(j * 16, 16)] = ga * a + gb * bv
        return carry

    lax.fori_loop(0, TPW, row_body, 0)
    pltpu.sync_copy(rows0_v, out_hbm.at[pl.ds(base, TPW), :])


def _combine(y, dst0, dst1, g0):
    mesh = plsc.VectorSubcoreMesh(core_axis_name="c", subcore_axis_name="s")
    f = pl.kernel(
        _combine_body,
        out_type=jax.ShapeDtypeStruct((NT, DM), jnp.float32),
        mesh=mesh,
        scratch_types=[
            pltpu.VMEM((TPW,), jnp.int32),
            pltpu.VMEM((TPW,), jnp.int32),
            pltpu.VMEM((TPW, 16), jnp.float32),
            pltpu.VMEM((TPW, DM), jnp.float32),
            pltpu.VMEM((TPW, DM), jnp.float32),
            pltpu.SemaphoreType.DMA,
            pltpu.SemaphoreType.DMA,
        ],
    )
    return f(y, dst0, dst1, g0)


def kernel(x, Wg, W1, b1, W2, b2):
    wgp = jnp.pad(Wg, ((0, 0), (0, LN - NE)))
    dst0b, dst1b, g0b, bexpb, nactb, lossb = _router(x, wgp)
    dst0 = dst0b[:, 0]
    dst1 = dst1b[:, 0]
    g0 = g0b[:, :16]
    bexp = bexpb[:NB, 0]
    nact = nactb[0, :1]
    loss = lossb[0, 0]

    xg = _dispatch(x, dst0, dst1)
    y = _experts(bexp, nact, xg, W1, b1, W2, b2)
    out = _combine(y, dst0, dst1, g0)
    return out, loss
